# baseline (speedup 1.0000x reference)
import jax
import jax.numpy as jnp
from jax.experimental import pallas as pl


def _body(h, w, keep_n, x_ref, o_ref):
    hw = h * w
    xb = x_ref[0]
    e = jnp.sum(xb * xb, axis=0, keepdims=True)
    lane = jax.lax.broadcasted_iota(jnp.int32, (h, hw), 1)
    rowi = jax.lax.broadcasted_iota(jnp.int32, (h, hw), 0)
    lrow = lane // w
    cond = lrow == rowi
    g = jnp.where(cond, jnp.broadcast_to(e, (h, hw)), -1.0)
    rm_col = jnp.max(g, axis=1, keepdims=True)
    rmb = jnp.sum(jnp.where(cond, jnp.broadcast_to(rm_col, (h, hw)), 0.0),
                  axis=0, keepdims=True)
    less = rmb < rm_col
    eq_lower = (rmb == rm_col) & (lrow < rowi)
    cnt = jnp.sum(less.astype(jnp.float32) + eq_lower.astype(jnp.float32),
                  axis=1, keepdims=True)
    rank = cnt * (1.0 / w)
    keep = (rank < keep_n).astype(xb.dtype)
    mask = jnp.sum(jnp.where(cond, jnp.broadcast_to(keep, (h, hw)), 0.0),
                   axis=0, keepdims=True)
    o_ref[0] = xb * mask


def kernel(x):
    b, c, h, w = x.shape
    rh = int(round(0.33 * h))
    keep_n = h - rh
    x3 = x.reshape(b, c, h * w)
    out = pl.pallas_call(
        lambda x_ref, o_ref: _body(h, w, keep_n, x_ref, o_ref),
        grid=(b,),
        in_specs=[pl.BlockSpec((1, c, h * w), lambda i: (i, 0, 0))],
        out_specs=pl.BlockSpec((1, c, h * w), lambda i: (i, 0, 0)),
        out_shape=jax.ShapeDtypeStruct((b, c, h * w), x.dtype),
    )(x3)
    return out.reshape(b, c, h, w)
